# Initial kernel scaffold; baseline (speedup 1.0000x reference)
import jax
import jax.numpy as jnp
from jax.experimental import pallas as pl


def kernel(x, points):
    raise NotImplementedError("write your pallas kernel here")


# speedup vs baseline: 841.3216x; 841.3216x over previous
import jax
import jax.numpy as jnp
from jax.experimental import pallas as pl

N_REGIONS = 6
BOUND = 2.5


def _pwlu_tc_kernel(x_ref, pts_ref, out_ref):
    x = x_ref[...]
    xn = x * (0.5 / BOUND) + 0.5
    scaled = xn * N_REGIONS
    r = jnp.floor(jnp.clip(xn, 0.0, 0.999) * N_REGIONS)
    d = scaled - r
    one_minus_d = 1.0 - d
    acc = jnp.zeros_like(x)
    for i in range(N_REGIONS):
        li = pts_ref[:, i][:, None]
        ri = pts_ref[:, i + 1][:, None]
        seg = li * one_minus_d + ri * d
        acc = jnp.where(r == float(i), seg, acc)
    out_ref[...] = acc


def kernel(x, points):
    B, C, H, W = x.shape
    n_pts = points.shape[1]
    rows = B * C
    cols = H * W
    x2 = x.reshape(rows, cols)
    pts = jnp.tile(points, (B, 1))

    ROW_BLK = 16
    grid = (rows // ROW_BLK,)
    out = pl.pallas_call(
        _pwlu_tc_kernel,
        grid=grid,
        in_specs=[
            pl.BlockSpec((ROW_BLK, cols), lambda i: (i, 0)),
            pl.BlockSpec((ROW_BLK, n_pts), lambda i: (i, 0)),
        ],
        out_specs=pl.BlockSpec((ROW_BLK, cols), lambda i: (i, 0)),
        out_shape=jax.ShapeDtypeStruct((rows, cols), x.dtype),
    )(x2, pts)
    return out.reshape(B, C, H, W)


# speedup vs baseline: 1184.9772x; 1.4085x over previous
import jax
import jax.numpy as jnp
from jax.experimental import pallas as pl

N_REGIONS = 6
BOUND = 2.5


def _pwlu_tc_kernel(x_ref, a_ref, b_ref, out_ref):
    x = x_ref[...]
    s = x * (0.5 * N_REGIONS / BOUND) + (0.5 * N_REGIONS)
    a = a_ref[:, 0][:, None]
    b = b_ref[:, 0][:, None]
    for j in range(1, N_REGIONS):
        m = s >= float(j)
        a = jnp.where(m, a_ref[:, j][:, None], a)
        b = jnp.where(m, b_ref[:, j][:, None], b)
    out_ref[...] = a + b * s


def kernel(x, points):
    B, C, H, W = x.shape
    rows = B * C
    cols = H * W
    x2 = x.reshape(rows, cols)

    slopes = points[:, 1:] - points[:, :-1]
    intercepts = points[:, :-1] - slopes * jnp.arange(
        N_REGIONS, dtype=points.dtype
    )[None, :]
    a_t = jnp.tile(intercepts, (B, 1))
    b_t = jnp.tile(slopes, (B, 1))

    ROW_BLK = 32
    grid = (rows // ROW_BLK,)
    out = pl.pallas_call(
        _pwlu_tc_kernel,
        grid=grid,
        in_specs=[
            pl.BlockSpec((ROW_BLK, cols), lambda i: (i, 0)),
            pl.BlockSpec((ROW_BLK, N_REGIONS), lambda i: (i, 0)),
            pl.BlockSpec((ROW_BLK, N_REGIONS), lambda i: (i, 0)),
        ],
        out_specs=pl.BlockSpec((ROW_BLK, cols), lambda i: (i, 0)),
        out_shape=jax.ShapeDtypeStruct((rows, cols), x.dtype),
    )(x2, a_t, b_t)
    return out.reshape(B, C, H, W)


# speedup vs baseline: 2615.6820x; 2.2074x over previous
import jax
import jax.numpy as jnp
from jax.experimental import pallas as pl

N_REGIONS = 6
BOUND = 2.5


def _pwlu_tc_kernel(x_ref, a_ref, b_ref, out_ref):
    x = x_ref[...]
    s = x * (0.5 * N_REGIONS / BOUND) + (0.5 * N_REGIONS)
    a = jnp.full_like(s, a_ref[0, 0, 0])
    b = jnp.full_like(s, b_ref[0, 0, 0])
    for j in range(1, N_REGIONS):
        m = s >= float(j)
        a = jnp.where(m, a_ref[0, 0, j], a)
        b = jnp.where(m, b_ref[0, 0, j], b)
    out_ref[...] = a + b * s


def kernel(x, points):
    B, C, H, W = x.shape

    slopes = points[:, 1:] - points[:, :-1]
    intercepts = points[:, :-1] - slopes * jnp.arange(
        N_REGIONS, dtype=points.dtype
    )[None, :]
    a_t = intercepts.reshape(C, 1, N_REGIONS)
    b_t = slopes.reshape(C, 1, N_REGIONS)

    grid = (C,)
    out = pl.pallas_call(
        _pwlu_tc_kernel,
        grid=grid,
        in_specs=[
            pl.BlockSpec((B, 1, H, W), lambda c: (0, c, 0, 0)),
            pl.BlockSpec((1, 1, N_REGIONS), lambda c: (c, 0, 0)),
            pl.BlockSpec((1, 1, N_REGIONS), lambda c: (c, 0, 0)),
        ],
        out_specs=pl.BlockSpec((B, 1, H, W), lambda c: (0, c, 0, 0)),
        out_shape=jax.ShapeDtypeStruct((B, C, H, W), x.dtype),
    )(x, a_t, b_t)
    return out
